# baseline (speedup 1.0000x reference)
import jax
import jax.numpy as jnp
from jax.experimental import pallas as pl
from jax.experimental.pallas import tpu as pltpu

D_IN_ = 768
D_HID_ = 3072
D_OUT_ = 768
E_ = 16
HC_ = 4
CH_ = D_HID_ // HC_


def _moe_kernel(xf_ref, wr_ref, br_ref, w1_ref, b1_ref, w2_ref, b2_ref,
                out_ref, route_ref):
    e = pl.program_id(0)
    c = pl.program_id(1)

    @pl.when((e == 0) & (c == 0))
    def _compute_routing():
        logits = jnp.dot(xf_ref[...], wr_ref[...],
                         preferred_element_type=jnp.float32)
        logits = logits + br_ref[...]
        n, ecnt = logits.shape
        lane = jax.lax.broadcasted_iota(jnp.int32, (n, ecnt), 1)
        neg_inf = jnp.float32(-jnp.inf)
        m1 = jnp.max(logits, axis=1, keepdims=True)
        i1 = jnp.min(jnp.where(logits == m1, lane, ecnt), axis=1, keepdims=True)
        masked = jnp.where(lane == i1, neg_inf, logits)
        m2 = jnp.max(masked, axis=1, keepdims=True)
        i2 = jnp.min(jnp.where(masked == m2, lane, ecnt), axis=1, keepdims=True)
        p1 = 1.0 / (1.0 + jnp.exp(m2 - m1))
        route_ref[:, 0:1] = i1.astype(jnp.float32)
        route_ref[:, 1:2] = i2.astype(jnp.float32)
        route_ref[:, 2:3] = p1
        route_ref[:, 3:4] = 1.0 - p1

    h = jnp.dot(xf_ref[...], w1_ref[0], preferred_element_type=jnp.float32)
    h = jnp.maximum(h + b1_ref[0, 0, 0], 0.0)
    y = jnp.dot(h, w2_ref[0], preferred_element_type=jnp.float32)

    ef = e.astype(jnp.float32)
    gate = (jnp.where(route_ref[:, 0:1] == ef, route_ref[:, 2:3], 0.0)
            + jnp.where(route_ref[:, 1:2] == ef, route_ref[:, 3:4], 0.0))

    @pl.when(c == 0)
    def _with_bias():
        contrib = gate * (y + b2_ref[0])

        @pl.when(e == 0)
        def _init():
            out_ref[...] = contrib

        @pl.when(e != 0)
        def _acc():
            out_ref[...] += contrib

    @pl.when(c != 0)
    def _no_bias():
        out_ref[...] += gate * y


@jax.jit
def kernel(x, Wr, br, W1, b1, W2, b2):
    Bsz, Ssz, d = x.shape
    xf = x.reshape(-1, d)
    n = xf.shape[0]
    out = pl.pallas_call(
        _moe_kernel,
        grid=(E_, HC_),
        in_specs=[
            pl.BlockSpec((n, D_IN_), lambda e, c: (0, 0)),
            pl.BlockSpec((D_IN_, E_), lambda e, c: (0, 0)),
            pl.BlockSpec((1, E_), lambda e, c: (0, 0)),
            pl.BlockSpec((1, D_IN_, CH_), lambda e, c: (e, 0, c)),
            pl.BlockSpec((1, 1, 1, CH_), lambda e, c: (e, c, 0, 0)),
            pl.BlockSpec((1, CH_, D_OUT_), lambda e, c: (e, c, 0)),
            pl.BlockSpec((1, 1, D_OUT_), lambda e, c: (e, 0, 0)),
        ],
        out_specs=pl.BlockSpec((n, D_OUT_), lambda e, c: (0, 0)),
        out_shape=jax.ShapeDtypeStruct((n, D_OUT_), jnp.float32),
        scratch_shapes=[pltpu.VMEM((n, 8), jnp.float32)],
    )(xf, Wr, br.reshape(1, E_), W1, b1.reshape(E_, HC_, 1, CH_), W2,
      b2.reshape(E_, 1, D_OUT_))
    return out.reshape(Bsz, Ssz, D_OUT_)


# speedup vs baseline: 1.1292x; 1.1292x over previous
import jax
import jax.numpy as jnp
from jax.experimental import pallas as pl
from jax.experimental.pallas import tpu as pltpu

D_IN_ = 768
D_HID_ = 3072
D_OUT_ = 768
E_ = 16


def _moe_kernel(xf_ref, wr_ref, br_ref, w1a_ref, w1b_ref, b1_ref,
                w2a_ref, w2b_ref, b2_ref, out_ref, route_ref):
    e = pl.program_id(0)

    @pl.when(e == 0)
    def _compute_routing():
        logits = jnp.dot(xf_ref[...], wr_ref[...],
                         preferred_element_type=jnp.float32)
        logits = logits + br_ref[...]
        n, ecnt = logits.shape
        lane = jax.lax.broadcasted_iota(jnp.int32, (n, ecnt), 1)
        neg_inf = jnp.float32(-jnp.inf)
        m1 = jnp.max(logits, axis=1, keepdims=True)
        i1 = jnp.min(jnp.where(logits == m1, lane, ecnt), axis=1, keepdims=True)
        masked = jnp.where(lane == i1, neg_inf, logits)
        m2 = jnp.max(masked, axis=1, keepdims=True)
        i2 = jnp.min(jnp.where(masked == m2, lane, ecnt), axis=1, keepdims=True)
        p1 = 1.0 / (1.0 + jnp.exp(m2 - m1))
        route_ref[:, 0:1] = i1.astype(jnp.float32)
        route_ref[:, 1:2] = i2.astype(jnp.float32)
        route_ref[:, 2:3] = p1
        route_ref[:, 3:4] = 1.0 - p1

    xf = xf_ref[...]
    half_in = D_IN_ // 2
    half_hid = D_HID_ // 2
    h = (jnp.dot(xf[:, :half_in], w1a_ref[0], preferred_element_type=jnp.float32)
         + jnp.dot(xf[:, half_in:], w1b_ref[0], preferred_element_type=jnp.float32))
    h = jnp.maximum(h + b1_ref[0], 0.0)
    y = (jnp.dot(h[:, :half_hid], w2a_ref[0], preferred_element_type=jnp.float32)
         + jnp.dot(h[:, half_hid:], w2b_ref[0], preferred_element_type=jnp.float32))
    y = y + b2_ref[0]

    ef = e.astype(jnp.float32)
    gate = (jnp.where(route_ref[:, 0:1] == ef, route_ref[:, 2:3], 0.0)
            + jnp.where(route_ref[:, 1:2] == ef, route_ref[:, 3:4], 0.0))
    contrib = gate * y

    @pl.when(e == 0)
    def _init():
        out_ref[...] = contrib

    @pl.when(e != 0)
    def _acc():
        out_ref[...] += contrib


@jax.jit
def kernel(x, Wr, br, W1, b1, W2, b2):
    Bsz, Ssz, d = x.shape
    xf = x.reshape(-1, d)
    n = xf.shape[0]
    half_in = D_IN_ // 2
    half_hid = D_HID_ // 2
    out = pl.pallas_call(
        _moe_kernel,
        grid=(E_,),
        in_specs=[
            pl.BlockSpec((n, D_IN_), lambda e: (0, 0)),
            pl.BlockSpec((D_IN_, E_), lambda e: (0, 0)),
            pl.BlockSpec((1, E_), lambda e: (0, 0)),
            pl.BlockSpec((1, half_in, D_HID_), lambda e: (e, 0, 0)),
            pl.BlockSpec((1, half_in, D_HID_), lambda e: (e, 1, 0)),
            pl.BlockSpec((1, 1, D_HID_), lambda e: (e, 0, 0)),
            pl.BlockSpec((1, half_hid, D_OUT_), lambda e: (e, 0, 0)),
            pl.BlockSpec((1, half_hid, D_OUT_), lambda e: (e, 1, 0)),
            pl.BlockSpec((1, 1, D_OUT_), lambda e: (e, 0, 0)),
        ],
        out_specs=pl.BlockSpec((n, D_OUT_), lambda e: (0, 0)),
        out_shape=jax.ShapeDtypeStruct((n, D_OUT_), jnp.float32),
        scratch_shapes=[pltpu.VMEM((n, 8), jnp.float32)],
    )(xf, Wr, br.reshape(1, E_), W1, W1, b1.reshape(E_, 1, D_HID_),
      W2, W2, b2.reshape(E_, 1, D_OUT_))
    return out.reshape(Bsz, Ssz, D_OUT_)
